# Initial kernel scaffold; baseline (speedup 1.0000x reference)
import jax
import jax.numpy as jnp
from jax.experimental import pallas as pl


def kernel(pc1, feat1, flow, cost, W_rw1, b_rw1, W_rw2, b_rw2, W_f1, b_f1, g_bn1, be_bn1, W_f2, b_f2, g_bn2, be_bn2, W_f3, b_f3, W_fc, b_fc):
    raise NotImplementedError("write your pallas kernel here")


# speedup vs baseline: 11.1787x; 11.1787x over previous
import functools

import jax
import jax.numpy as jnp
from jax import lax
from jax.experimental import pallas as pl
from jax.experimental.pallas import tpu as pltpu
from jax.experimental.pallas import tpu_sc as plsc

NK = 16
BN_EPS = 1e-5
BIG = 3.0e38


def _precompute_body(x_ref, wt_ref, wcb_ref, t_ref, cb_ref):
    x = x_ref[...]
    t_ref[...] = jnp.dot(x, wt_ref[...], preferred_element_type=jnp.float32)
    cb_ref[...] = jnp.dot(x, wcb_ref[...], preferred_element_type=jnp.float32)


def _topk_body(rows_ref, cols_ref, out_ref, d_ref, *, tn, n, k):
    b = pl.program_id(0)
    rows = rows_ref[0]
    cols = cols_ref[0]
    colsq = jnp.sum(cols * cols, axis=0, keepdims=True)
    d_ref[...] = colsq - 2.0 * jnp.dot(rows, cols,
                                       preferred_element_type=jnp.float32)
    iota_c = lax.broadcasted_iota(jnp.int32, (tn, n), 1)
    iota_k = lax.broadcasted_iota(jnp.int32, (tn, k), 1)

    def body(t, obuf):
        d = d_ref[...]
        m = jnp.min(d, axis=1, keepdims=True)
        sel = jnp.where(d <= m, iota_c, jnp.int32(n))
        idxv = jnp.min(sel, axis=1, keepdims=True)
        obuf = jnp.where(iota_k == t, idxv, obuf)
        d_ref[...] = jnp.where(iota_c == idxv, BIG, d)
        return obuf

    obuf = lax.fori_loop(0, k, body, jnp.zeros((tn, k), jnp.int32))
    out_ref[0] = obuf + b * n


def _stats1_body(g_ref, cb_ref, s_ref, *, tnn, k, c1):
    h = g_ref[...][:, :, :c1] + cb_ref[...][:, None, :c1]
    hr = h.reshape(tnn * k, c1)
    st = jnp.stack([jnp.sum(hr, axis=0), jnp.sum(hr * hr, axis=0)])

    @pl.when(pl.program_id(0) == 0)
    def _():
        s_ref[...] = st

    @pl.when(pl.program_id(0) != 0)
    def _():
        s_ref[...] += st


def _bn_affine(s_ref, g_ref, be_ref, cnt):
    mu = s_ref[0:1, :] * (1.0 / cnt)
    var = s_ref[1:2, :] * (1.0 / cnt) - mu * mu
    a = g_ref[...] * lax.rsqrt(var + BN_EPS)
    c = be_ref[...] - mu * a
    return a, c


def _m2_body(g_ref, cb_ref, s_ref, g1_ref, be1_ref, w2_ref, b2_ref,
             h2_ref, s2_ref, *, tnn, k, c1, cnt):
    a, c = _bn_affine(s_ref, g1_ref, be1_ref, cnt)
    h = g_ref[...][:, :, :c1] + cb_ref[...][:, None, :c1]
    y = a * h.reshape(tnn * k, c1) + c
    y = jnp.where(y >= 0, y, 0.1 * y)
    h2 = jnp.dot(y, w2_ref[...], preferred_element_type=jnp.float32)
    h2 = h2 + b2_ref[...]
    h2_ref[...] = h2
    st = jnp.stack([jnp.sum(h2, axis=0), jnp.sum(h2 * h2, axis=0)])

    @pl.when(pl.program_id(0) == 0)
    def _():
        s2_ref[...] = st

    @pl.when(pl.program_id(0) != 0)
    def _():
        s2_ref[...] += st


def _m3_body(h2_ref, g_ref, cb_ref, s_ref, g2_ref, be2_ref, w3_ref, b3_ref,
             w2a_ref, w2b_ref, brw2_ref, wfc_ref, bfc_ref, fl_ref, o_ref,
             *, tnn, k, c1, cnt):
    a, c = _bn_affine(s_ref, g2_ref, be2_ref, cnt)
    y = a * h2_ref[...] + c
    y = jnp.where(y >= 0, y, 0.1 * y)
    reflow = jnp.dot(y, w3_ref[...],
                     preferred_element_type=jnp.float32) + b3_ref[...]
    reflow3 = reflow.reshape(tnn, k, reflow.shape[-1])
    rew3 = g_ref[...][:, :, c1:] + cb_ref[...][:, None, c1:]
    logits = jnp.sum(rew3 * w2a_ref[...] + reflow3 * w2b_ref[...], axis=2)
    logits = logits + brw2_ref[...]
    m = jnp.max(logits, axis=1, keepdims=True)
    e = jnp.exp(logits - m)
    w = e / jnp.sum(e, axis=1, keepdims=True)
    rwf = jnp.sum(reflow3 * w[:, :, None], axis=1)
    o = jnp.dot(rwf, wfc_ref[...],
                preferred_element_type=jnp.float32) + bfc_ref[...]
    o_ref[...] = o + fl_ref[...]


def _make_sc_gather(bnk, width, ch):
    nw = 32
    rpw = bnk // nw
    nch = rpw // ch
    mesh = plsc.VectorSubcoreMesh(core_axis_name="c", subcore_axis_name="s")

    @functools.partial(
        pl.kernel,
        mesh=mesh,
        out_type=jax.ShapeDtypeStruct((bnk, width), jnp.float32),
        scratch_types=[
            pltpu.VMEM((ch,), jnp.int32),
            pltpu.VMEM((ch, width), jnp.float32),
            pltpu.SemaphoreType.DMA,
        ],
        compiler_params=pltpu.CompilerParams(use_tc_tiling_on_sc=False),
    )
    def sc_gather(tbl_hbm, idx_hbm, out_hbm, idx_v, rows_v, sem):
        wid = lax.axis_index("s") * 2 + lax.axis_index("c")

        def body(cstep, carry):
            base = pl.multiple_of(wid * rpw + cstep * ch, ch)
            pltpu.sync_copy(idx_hbm.at[pl.ds(base, ch)], idx_v)
            pltpu.async_copy(tbl_hbm.at[idx_v], rows_v, sem).wait()
            pltpu.sync_copy(rows_v, out_hbm.at[pl.ds(base, ch)])
            return carry

        lax.fori_loop(0, nch, body, 0)

    return sc_gather


def kernel(pc1, feat1, flow, cost, W_rw1, b_rw1, W_rw2, b_rw2, W_f1, b_f1,
           g_bn1, be_bn1, W_f2, b_f2, g_bn2, be_bn2, W_f3, b_f3, W_fc, b_fc):
    f32 = jnp.float32
    B, _, N = pc1.shape
    D = feat1.shape[1]
    CC = cost.shape[1]
    K = NK
    BNr = B * N
    BNK = BNr * K
    C1 = W_f1.shape[0]
    R = W_rw1.shape[0]
    C2 = W_f2.shape[0]
    C3 = W_f3.shape[0]
    WIDTH = C1 + R

    pc1t = pc1.transpose(0, 2, 1).reshape(BNr, 3)
    feat1t = feat1.transpose(0, 2, 1).reshape(BNr, D)
    flowt = flow.transpose(0, 2, 1).reshape(BNr, 3)
    costt = cost.transpose(0, 2, 1).reshape(BNr, CC)
    XIN = 3 + D + 3 + CC + 1
    XPAD = 256
    X = jnp.concatenate(
        [pc1t, feat1t, flowt, costt, jnp.ones((BNr, 1), f32),
         jnp.zeros((BNr, XPAD - XIN), f32)], axis=1)

    Wx = W_f1[:, 0:3]
    Wgf = W_f1[:, 3:3 + D]
    Wcf = W_f1[:, 3 + D:3 + 2 * D]
    Wgw = W_f1[:, 3 + 2 * D:6 + 2 * D]
    Wcw = W_f1[:, 6 + 2 * D:9 + 2 * D]
    Wgc = W_f1[:, 9 + 2 * D:9 + 2 * D + CC]
    A_rw = W_rw1[:, 0:3] + W_rw1[:, 3:6]
    B_rw = W_rw1[:, 6:9] - W_rw1[:, 0:3]
    zR = jnp.zeros((D, R), f32)
    z3R = jnp.zeros((3, R), f32)
    W_T = jnp.concatenate([
        jnp.concatenate([Wx.T, A_rw.T], axis=1),
        jnp.concatenate([Wgf.T, zR], axis=1),
        jnp.concatenate([Wgw.T, z3R], axis=1),
        jnp.concatenate([Wgc.T, zR], axis=1),
        jnp.zeros((1 + XPAD - XIN, WIDTH), f32),
    ], axis=0)
    W_CB = jnp.concatenate([
        jnp.concatenate([-Wx.T, B_rw.T], axis=1),
        jnp.concatenate([Wcf.T, zR], axis=1),
        jnp.concatenate([Wcw.T, z3R], axis=1),
        jnp.zeros((CC, WIDTH), f32),
        jnp.concatenate([b_f1[None, :], b_rw1[None, :]], axis=1),
        jnp.zeros((XPAD - XIN, WIDTH), f32),
    ], axis=0)

    PT = 512
    T, CB = pl.pallas_call(
        _precompute_body,
        grid=(BNr // PT,),
        in_specs=[
            pl.BlockSpec((PT, XPAD), lambda i: (i, 0)),
            pl.BlockSpec((XPAD, WIDTH), lambda i: (0, 0)),
            pl.BlockSpec((XPAD, WIDTH), lambda i: (0, 0)),
        ],
        out_specs=[
            pl.BlockSpec((PT, WIDTH), lambda i: (i, 0)),
            pl.BlockSpec((PT, WIDTH), lambda i: (i, 0)),
        ],
        out_shape=[
            jax.ShapeDtypeStruct((BNr, WIDTH), f32),
            jax.ShapeDtypeStruct((BNr, WIDTH), f32),
        ],
    )(X, W_T, W_CB)

    TN = 128
    pc3 = pc1t.reshape(B, N, 3)
    kidx = pl.pallas_call(
        functools.partial(_topk_body, tn=TN, n=N, k=K),
        grid=(B, N // TN),
        in_specs=[
            pl.BlockSpec((1, TN, 3), lambda b, i: (b, i, 0)),
            pl.BlockSpec((1, 3, N), lambda b, i: (b, 0, 0)),
        ],
        out_specs=pl.BlockSpec((1, TN, K), lambda b, i: (b, i, 0)),
        out_shape=jax.ShapeDtypeStruct((B, N, K), jnp.int32),
        scratch_shapes=[pltpu.VMEM((TN, N), f32)],
    )(pc3, pc1)

    CH = 128
    G = _make_sc_gather(BNK, WIDTH, CH)(T, kidx.reshape(BNK))
    G3 = G.reshape(BNr, K, WIDTH)

    TNN = 128
    grid_r = BNr // TNN
    g_spec = pl.BlockSpec((TNN, K, WIDTH), lambda i: (i, 0, 0))
    cb_spec = pl.BlockSpec((TNN, WIDTH), lambda i: (i, 0))
    s1_spec = pl.BlockSpec((2, C1), lambda i: (0, 0))
    stats1 = pl.pallas_call(
        functools.partial(_stats1_body, tnn=TNN, k=K, c1=C1),
        grid=(grid_r,),
        in_specs=[g_spec, cb_spec],
        out_specs=s1_spec,
        out_shape=jax.ShapeDtypeStruct((2, C1), f32),
    )(G3, CB)

    s2_spec = pl.BlockSpec((2, C2), lambda i: (0, 0))
    h2, stats2 = pl.pallas_call(
        functools.partial(_m2_body, tnn=TNN, k=K, c1=C1, cnt=float(BNK)),
        grid=(grid_r,),
        in_specs=[
            g_spec, cb_spec, s1_spec,
            pl.BlockSpec((1, C1), lambda i: (0, 0)),
            pl.BlockSpec((1, C1), lambda i: (0, 0)),
            pl.BlockSpec((C1, C2), lambda i: (0, 0)),
            pl.BlockSpec((1, C2), lambda i: (0, 0)),
        ],
        out_specs=[
            pl.BlockSpec((TNN * K, C2), lambda i: (i, 0)),
            s2_spec,
        ],
        out_shape=[
            jax.ShapeDtypeStruct((BNK, C2), f32),
            jax.ShapeDtypeStruct((2, C2), f32),
        ],
    )(G3, CB, stats1, g_bn1.reshape(1, C1), be_bn1.reshape(1, C1),
      W_f2.T, b_f2.reshape(1, C2))

    outp = pl.pallas_call(
        functools.partial(_m3_body, tnn=TNN, k=K, c1=C1, cnt=float(BNK)),
        grid=(grid_r,),
        in_specs=[
            pl.BlockSpec((TNN * K, C2), lambda i: (i, 0)),
            g_spec, cb_spec, s2_spec,
            pl.BlockSpec((1, C2), lambda i: (0, 0)),
            pl.BlockSpec((1, C2), lambda i: (0, 0)),
            pl.BlockSpec((C2, C3), lambda i: (0, 0)),
            pl.BlockSpec((1, C3), lambda i: (0, 0)),
            pl.BlockSpec((1, 1, R), lambda i: (0, 0, 0)),
            pl.BlockSpec((1, 1, C3), lambda i: (0, 0, 0)),
            pl.BlockSpec((1, 1), lambda i: (0, 0)),
            pl.BlockSpec((C3, 3), lambda i: (0, 0)),
            pl.BlockSpec((1, 3), lambda i: (0, 0)),
            pl.BlockSpec((TNN, 3), lambda i: (i, 0)),
        ],
        out_specs=pl.BlockSpec((TNN, 3), lambda i: (i, 0)),
        out_shape=jax.ShapeDtypeStruct((BNr, 3), f32),
    )(h2, G3, CB, stats2, g_bn2.reshape(1, C2), be_bn2.reshape(1, C2),
      W_f3.T, b_f3.reshape(1, C3), W_rw2[:, :R].reshape(1, 1, R),
      W_rw2[:, R:].reshape(1, 1, C3), b_rw2.reshape(1, 1), W_fc.T,
      b_fc.reshape(1, 3), flowt)

    return outp.reshape(B, N, 3).transpose(0, 2, 1)


# speedup vs baseline: 12.6601x; 1.1325x over previous
import functools

import jax
import jax.numpy as jnp
from jax import lax
from jax.experimental import pallas as pl
from jax.experimental.pallas import tpu as pltpu
from jax.experimental.pallas import tpu_sc as plsc

NK = 16
BN_EPS = 1e-5
BIG = 3.0e38


def _precompute_body(x_ref, wtu_ref, wta_ref, wcu_ref, wca_ref,
                     tu_ref, ta_ref, cu_ref, ca_ref):
    x = x_ref[...]
    tu_ref[...] = jnp.dot(x, wtu_ref[...], preferred_element_type=jnp.float32)
    ta_ref[...] = jnp.dot(x, wta_ref[...], preferred_element_type=jnp.float32)
    cu_ref[...] = jnp.dot(x, wcu_ref[...], preferred_element_type=jnp.float32)
    ca_ref[...] = jnp.dot(x, wca_ref[...], preferred_element_type=jnp.float32)


def _topk_body(rows_ref, cols_ref, out_ref, d_ref, *, tn, n, k):
    b = pl.program_id(0)
    rows = rows_ref[0]
    cols = cols_ref[0]
    colsq = jnp.sum(cols * cols, axis=0, keepdims=True)
    d_ref[...] = colsq - 2.0 * jnp.dot(rows, cols,
                                       preferred_element_type=jnp.float32)
    iota_c = lax.broadcasted_iota(jnp.int32, (tn, n), 1)
    iota_k = lax.broadcasted_iota(jnp.int32, (tn, k), 1)

    def body(t, carry):
        v, obuf = carry
        d = d_ref[...]
        idxv = jnp.min(jnp.where(d == v, iota_c, jnp.int32(n)),
                       axis=1, keepdims=True)
        obuf = jnp.where(iota_k == t - 1, idxv, obuf)
        v = jnp.min(jnp.where(d > v, d, BIG), axis=1, keepdims=True)
        return v, obuf

    _, obuf = lax.fori_loop(
        0, k + 1, body,
        (jnp.full((tn, 1), -BIG, jnp.float32), jnp.zeros((tn, k), jnp.int32)))
    out_ref[0] = obuf + b * n


def _stats1_body(g_ref, cb_ref, s_ref, *, tnn, k, c1):
    h = g_ref[...] + cb_ref[...][:, None, :]
    hr = h.reshape(tnn * k, c1)
    st = jnp.stack([jnp.sum(hr, axis=0), jnp.sum(hr * hr, axis=0)])

    @pl.when(pl.program_id(0) == 0)
    def _():
        s_ref[...] = st

    @pl.when(pl.program_id(0) != 0)
    def _():
        s_ref[...] += st


def _bn_affine(s_ref, g_ref, be_ref, cnt):
    mu = s_ref[0:1, :] * (1.0 / cnt)
    var = s_ref[1:2, :] * (1.0 / cnt) - mu * mu
    a = g_ref[...] * lax.rsqrt(var + BN_EPS)
    c = be_ref[...] - mu * a
    return a, c


def _m2_body(g_ref, cb_ref, s_ref, g1_ref, be1_ref, w2_ref, b2_ref,
             h2_ref, s2_ref, *, tnn, k, c1, cnt):
    a, c = _bn_affine(s_ref, g1_ref, be1_ref, cnt)
    h = g_ref[...] + cb_ref[...][:, None, :]
    y = a * h.reshape(tnn * k, c1) + c
    y = jnp.where(y >= 0, y, 0.1 * y)
    h2 = jnp.dot(y, w2_ref[...], preferred_element_type=jnp.float32)
    h2 = h2 + b2_ref[...]
    h2_ref[...] = h2
    st = jnp.stack([jnp.sum(h2, axis=0), jnp.sum(h2 * h2, axis=0)])

    @pl.when(pl.program_id(0) == 0)
    def _():
        s2_ref[...] = st

    @pl.when(pl.program_id(0) != 0)
    def _():
        s2_ref[...] += st


def _m3_body(h2_ref, g_ref, cb_ref, s_ref, g2_ref, be2_ref, w3_ref, b3_ref,
             w2a_ref, w2b_ref, brw2_ref, wfc_ref, bfc_ref, fl_ref, o_ref,
             *, tnn, k, c1, cnt):
    a, c = _bn_affine(s_ref, g2_ref, be2_ref, cnt)
    y = a * h2_ref[...] + c
    y = jnp.where(y >= 0, y, 0.1 * y)
    reflow = jnp.dot(y, w3_ref[...],
                     preferred_element_type=jnp.float32) + b3_ref[...]
    reflow3 = reflow.reshape(tnn, k, reflow.shape[-1])
    rew3 = g_ref[...] + cb_ref[...][:, None, :]
    logits = jnp.sum(rew3 * w2a_ref[...] + reflow3 * w2b_ref[...], axis=2)
    logits = logits + brw2_ref[...]
    m = jnp.max(logits, axis=1, keepdims=True)
    e = jnp.exp(logits - m)
    w = e / jnp.sum(e, axis=1, keepdims=True)
    rwf = jnp.sum(reflow3 * w[:, :, None], axis=1)
    o = jnp.dot(rwf, wfc_ref[...],
                preferred_element_type=jnp.float32) + bfc_ref[...]
    o_ref[...] = o + fl_ref[...]


def _make_sc_gather(bnk, wu, wa, ch):
    nw = 32
    rpw = bnk // nw
    nch = rpw // ch
    mesh = plsc.VectorSubcoreMesh(core_axis_name="c", subcore_axis_name="s")

    @functools.partial(
        pl.kernel,
        mesh=mesh,
        out_type=(jax.ShapeDtypeStruct((bnk, wu), jnp.float32),
                  jax.ShapeDtypeStruct((bnk, wa), jnp.float32)),
        scratch_types=[
            pltpu.VMEM((ch,), jnp.int32),
            pltpu.VMEM((ch, wu), jnp.float32),
            pltpu.VMEM((ch, wa), jnp.float32),
            pltpu.SemaphoreType.DMA,
        ],
        compiler_params=pltpu.CompilerParams(use_tc_tiling_on_sc=False),
    )
    def sc_gather(tu_hbm, ta_hbm, idx_hbm, outu_hbm, outa_hbm,
                  idx_v, rowsu_v, rowsa_v, sem):
        wid = lax.axis_index("s") * 2 + lax.axis_index("c")

        def body(cstep, carry):
            base = pl.multiple_of(wid * rpw + cstep * ch, ch)
            pltpu.sync_copy(idx_hbm.at[pl.ds(base, ch)], idx_v)
            cu = pltpu.async_copy(tu_hbm.at[idx_v], rowsu_v, sem)
            ca = pltpu.async_copy(ta_hbm.at[idx_v], rowsa_v, sem)
            cu.wait()
            ca.wait()
            pltpu.sync_copy(rowsu_v, outu_hbm.at[pl.ds(base, ch)])
            pltpu.sync_copy(rowsa_v, outa_hbm.at[pl.ds(base, ch)])
            return carry

        lax.fori_loop(0, nch, body, 0)

    return sc_gather


def kernel(pc1, feat1, flow, cost, W_rw1, b_rw1, W_rw2, b_rw2, W_f1, b_f1,
           g_bn1, be_bn1, W_f2, b_f2, g_bn2, be_bn2, W_f3, b_f3, W_fc, b_fc):
    f32 = jnp.float32
    B, _, N = pc1.shape
    D = feat1.shape[1]
    CC = cost.shape[1]
    K = NK
    BNr = B * N
    BNK = BNr * K
    C1 = W_f1.shape[0]
    R = W_rw1.shape[0]
    C2 = W_f2.shape[0]
    C3 = W_f3.shape[0]
    WIDTH = C1 + R

    pc1t = pc1.transpose(0, 2, 1).reshape(BNr, 3)
    feat1t = feat1.transpose(0, 2, 1).reshape(BNr, D)
    flowt = flow.transpose(0, 2, 1).reshape(BNr, 3)
    costt = cost.transpose(0, 2, 1).reshape(BNr, CC)
    XIN = 3 + D + 3 + CC + 1
    XPAD = 256
    X = jnp.concatenate(
        [pc1t, feat1t, flowt, costt, jnp.ones((BNr, 1), f32),
         jnp.zeros((BNr, XPAD - XIN), f32)], axis=1)

    Wx = W_f1[:, 0:3]
    Wgf = W_f1[:, 3:3 + D]
    Wcf = W_f1[:, 3 + D:3 + 2 * D]
    Wgw = W_f1[:, 3 + 2 * D:6 + 2 * D]
    Wcw = W_f1[:, 6 + 2 * D:9 + 2 * D]
    Wgc = W_f1[:, 9 + 2 * D:9 + 2 * D + CC]
    A_rw = W_rw1[:, 0:3] + W_rw1[:, 3:6]
    B_rw = W_rw1[:, 6:9] - W_rw1[:, 0:3]
    W_TU = jnp.concatenate([
        Wx.T, Wgf.T, Wgw.T, Wgc.T, jnp.zeros((1 + XPAD - XIN, C1), f32),
    ], axis=0)
    W_TA = jnp.concatenate([
        A_rw.T, jnp.zeros((XPAD - 3, R), f32),
    ], axis=0)
    W_CU = jnp.concatenate([
        -Wx.T, Wcf.T, Wcw.T, jnp.zeros((CC, C1), f32), b_f1[None, :],
        jnp.zeros((XPAD - XIN, C1), f32),
    ], axis=0)
    W_CA = jnp.concatenate([
        B_rw.T, jnp.zeros((XIN - 4, R), f32), b_rw1[None, :],
        jnp.zeros((XPAD - XIN, R), f32),
    ], axis=0)

    PT = 512
    full = lambda i: (0, 0)
    TU, TA, CU, CA = pl.pallas_call(
        _precompute_body,
        grid=(BNr // PT,),
        in_specs=[
            pl.BlockSpec((PT, XPAD), lambda i: (i, 0)),
            pl.BlockSpec((XPAD, C1), full),
            pl.BlockSpec((XPAD, R), full),
            pl.BlockSpec((XPAD, C1), full),
            pl.BlockSpec((XPAD, R), full),
        ],
        out_specs=[
            pl.BlockSpec((PT, C1), lambda i: (i, 0)),
            pl.BlockSpec((PT, R), lambda i: (i, 0)),
            pl.BlockSpec((PT, C1), lambda i: (i, 0)),
            pl.BlockSpec((PT, R), lambda i: (i, 0)),
        ],
        out_shape=[
            jax.ShapeDtypeStruct((BNr, C1), f32),
            jax.ShapeDtypeStruct((BNr, R), f32),
            jax.ShapeDtypeStruct((BNr, C1), f32),
            jax.ShapeDtypeStruct((BNr, R), f32),
        ],
    )(X, W_TU, W_TA, W_CU, W_CA)

    TN = 128
    pc3 = pc1t.reshape(B, N, 3)
    kidx = pl.pallas_call(
        functools.partial(_topk_body, tn=TN, n=N, k=K),
        grid=(B, N // TN),
        in_specs=[
            pl.BlockSpec((1, TN, 3), lambda b, i: (b, i, 0)),
            pl.BlockSpec((1, 3, N), lambda b, i: (b, 0, 0)),
        ],
        out_specs=pl.BlockSpec((1, TN, K), lambda b, i: (b, i, 0)),
        out_shape=jax.ShapeDtypeStruct((B, N, K), jnp.int32),
        scratch_shapes=[pltpu.VMEM((TN, N), f32)],
    )(pc3, pc1)

    CH = 128
    GU, GA = _make_sc_gather(BNK, C1, R, CH)(TU, TA, kidx.reshape(BNK))
    GU3 = GU.reshape(BNr, K, C1)
    GA3 = GA.reshape(BNr, K, R)

    TNN = 128
    grid_r = BNr // TNN
    gu_spec = pl.BlockSpec((TNN, K, C1), lambda i: (i, 0, 0))
    ga_spec = pl.BlockSpec((TNN, K, R), lambda i: (i, 0, 0))
    cu_spec = pl.BlockSpec((TNN, C1), lambda i: (i, 0))
    ca_spec = pl.BlockSpec((TNN, R), lambda i: (i, 0))
    s1_spec = pl.BlockSpec((2, C1), lambda i: (0, 0))
    stats1 = pl.pallas_call(
        functools.partial(_stats1_body, tnn=TNN, k=K, c1=C1),
        grid=(grid_r,),
        in_specs=[gu_spec, cu_spec],
        out_specs=s1_spec,
        out_shape=jax.ShapeDtypeStruct((2, C1), f32),
    )(GU3, CU)

    s2_spec = pl.BlockSpec((2, C2), lambda i: (0, 0))
    h2, stats2 = pl.pallas_call(
        functools.partial(_m2_body, tnn=TNN, k=K, c1=C1, cnt=float(BNK)),
        grid=(grid_r,),
        in_specs=[
            gu_spec, cu_spec, s1_spec,
            pl.BlockSpec((1, C1), full),
            pl.BlockSpec((1, C1), full),
            pl.BlockSpec((C1, C2), full),
            pl.BlockSpec((1, C2), full),
        ],
        out_specs=[
            pl.BlockSpec((TNN * K, C2), lambda i: (i, 0)),
            s2_spec,
        ],
        out_shape=[
            jax.ShapeDtypeStruct((BNK, C2), f32),
            jax.ShapeDtypeStruct((2, C2), f32),
        ],
    )(GU3, CU, stats1, g_bn1.reshape(1, C1), be_bn1.reshape(1, C1),
      W_f2.T, b_f2.reshape(1, C2))

    outp = pl.pallas_call(
        functools.partial(_m3_body, tnn=TNN, k=K, c1=C1, cnt=float(BNK)),
        grid=(grid_r,),
        in_specs=[
            pl.BlockSpec((TNN * K, C2), lambda i: (i, 0)),
            ga_spec, ca_spec, s2_spec,
            pl.BlockSpec((1, C2), full),
            pl.BlockSpec((1, C2), full),
            pl.BlockSpec((C2, C3), full),
            pl.BlockSpec((1, C3), full),
            pl.BlockSpec((1, 1, R), lambda i: (0, 0, 0)),
            pl.BlockSpec((1, 1, C3), lambda i: (0, 0, 0)),
            pl.BlockSpec((1, 1), full),
            pl.BlockSpec((C3, 3), full),
            pl.BlockSpec((1, 3), full),
            pl.BlockSpec((TNN, 3), lambda i: (i, 0)),
        ],
        out_specs=pl.BlockSpec((TNN, 3), lambda i: (i, 0)),
        out_shape=jax.ShapeDtypeStruct((BNr, 3), f32),
    )(h2, GA3, CA, stats2, g_bn2.reshape(1, C2), be_bn2.reshape(1, C2),
      W_f3.T, b_f3.reshape(1, C3), W_rw2[:, :R].reshape(1, 1, R),
      W_rw2[:, R:].reshape(1, 1, C3), b_rw2.reshape(1, 1), W_fc.T,
      b_fc.reshape(1, 3), flowt)

    return outp.reshape(B, N, 3).transpose(0, 2, 1)


# speedup vs baseline: 13.2018x; 1.0428x over previous
import functools

import jax
import jax.numpy as jnp
from jax import lax
from jax.experimental import pallas as pl
from jax.experimental.pallas import tpu as pltpu
from jax.experimental.pallas import tpu_sc as plsc

NK = 16
BN_EPS = 1e-5
BIG = 3.0e38


def _precompute_body(x_ref, wtu_ref, wta_ref, wcu_ref, wca_ref,
                     tu_ref, ta_ref, cu_ref, ca_ref):
    x = x_ref[...]
    tu_ref[...] = jnp.dot(x, wtu_ref[...], preferred_element_type=jnp.float32)
    ta_ref[...] = jnp.dot(x, wta_ref[...], preferred_element_type=jnp.float32)
    cu_ref[...] = jnp.dot(x, wcu_ref[...], preferred_element_type=jnp.float32)
    ca_ref[...] = jnp.dot(x, wca_ref[...], preferred_element_type=jnp.float32)


def _topk_body(rows_ref, cols_ref, out_ref, d_ref, *, tn, n, k):
    b = pl.program_id(0)
    rows = rows_ref[0]
    cols = cols_ref[0]
    colsq = jnp.sum(cols * cols, axis=0, keepdims=True)
    d_ref[...] = colsq - 2.0 * jnp.dot(rows, cols,
                                       preferred_element_type=jnp.float32)
    iota_c = lax.broadcasted_iota(jnp.int32, (tn, n), 1)
    iota_k = lax.broadcasted_iota(jnp.int32, (tn, k), 1)

    def body(t, carry):
        v, obuf = carry
        d = d_ref[...]
        idxv = jnp.min(jnp.where(d == v, iota_c, jnp.int32(n)),
                       axis=1, keepdims=True)
        obuf = jnp.where(iota_k == t - 1, idxv, obuf)
        v = jnp.min(jnp.where(d > v, d, BIG), axis=1, keepdims=True)
        return v, obuf

    _, obuf = lax.fori_loop(
        0, k + 1, body,
        (jnp.full((tn, 1), -BIG, jnp.float32), jnp.zeros((tn, k), jnp.int32)))
    out_ref[0] = obuf + b * n


def _stats1_body(g_ref, cb_ref, s_ref, *, tnn, k, c1):
    h = g_ref[...] + cb_ref[...][:, None, :]
    hr = h.reshape(tnn * k, c1)
    st = jnp.stack([jnp.sum(hr, axis=0), jnp.sum(hr * hr, axis=0)])

    @pl.when(pl.program_id(0) == 0)
    def _():
        s_ref[...] = st

    @pl.when(pl.program_id(0) != 0)
    def _():
        s_ref[...] += st


def _bn_affine(s_ref, g_ref, be_ref, cnt):
    mu = s_ref[0:1, :] * (1.0 / cnt)
    var = s_ref[1:2, :] * (1.0 / cnt) - mu * mu
    a = g_ref[...] * lax.rsqrt(var + BN_EPS)
    c = be_ref[...] - mu * a
    return a, c


def _m2_body(g_ref, cb_ref, s_ref, g1_ref, be1_ref, w2_ref, b2_ref,
             h2_ref, s2_ref, *, tnn, k, c1, cnt):
    a, c = _bn_affine(s_ref, g1_ref, be1_ref, cnt)
    h = g_ref[...] + cb_ref[...][:, None, :]
    y = a * h.reshape(tnn * k, c1) + c
    y = jnp.where(y >= 0, y, 0.1 * y)
    h2 = jnp.dot(y, w2_ref[...], preferred_element_type=jnp.float32)
    h2 = h2 + b2_ref[...]
    h2_ref[...] = h2
    st = jnp.stack([jnp.sum(h2, axis=0), jnp.sum(h2 * h2, axis=0)])

    @pl.when(pl.program_id(0) == 0)
    def _():
        s2_ref[...] = st

    @pl.when(pl.program_id(0) != 0)
    def _():
        s2_ref[...] += st


def _m3_body(h2_ref, g_ref, cb_ref, s_ref, g2_ref, be2_ref, w3_ref, b3_ref,
             w2a_ref, w2b_ref, brw2_ref, wfc_ref, bfc_ref, fl_ref, o_ref,
             *, tnn, k, c1, cnt):
    a, c = _bn_affine(s_ref, g2_ref, be2_ref, cnt)
    y = a * h2_ref[...] + c
    y = jnp.where(y >= 0, y, 0.1 * y)
    reflow = jnp.dot(y, w3_ref[...],
                     preferred_element_type=jnp.float32) + b3_ref[...]
    reflow3 = reflow.reshape(tnn, k, reflow.shape[-1])
    rew3 = g_ref[...] + cb_ref[...][:, None, :]
    logits = jnp.sum(rew3 * w2a_ref[...] + reflow3 * w2b_ref[...], axis=2)
    logits = logits + brw2_ref[...]
    m = jnp.max(logits, axis=1, keepdims=True)
    e = jnp.exp(logits - m)
    w = e / jnp.sum(e, axis=1, keepdims=True)
    rwf = jnp.sum(reflow3 * w[:, :, None], axis=1)
    o = jnp.dot(rwf, wfc_ref[...],
                preferred_element_type=jnp.float32) + bfc_ref[...]
    o_ref[...] = o + fl_ref[...]


def _make_sc_gather(bnk, wu, wa, ch):
    nw = 32
    rpw = bnk // nw
    nch = rpw // ch
    mesh = plsc.VectorSubcoreMesh(core_axis_name="c", subcore_axis_name="s")

    @functools.partial(
        pl.kernel,
        mesh=mesh,
        out_type=(jax.ShapeDtypeStruct((bnk, wu), jnp.float32),
                  jax.ShapeDtypeStruct((bnk, wa), jnp.float32)),
        scratch_types=[
            pltpu.VMEM((ch,), jnp.int32),
            pltpu.VMEM((ch, wu), jnp.float32),
            pltpu.VMEM((ch, wa), jnp.float32),
            pltpu.SemaphoreType.DMA,
        ],
        compiler_params=pltpu.CompilerParams(use_tc_tiling_on_sc=False),
    )
    def sc_gather(tu_hbm, ta_hbm, idx_hbm, outu_hbm, outa_hbm,
                  idx_v, rowsu_v, rowsa_v, sem):
        wid = lax.axis_index("s") * 2 + lax.axis_index("c")

        def body(cstep, carry):
            base = pl.multiple_of(wid * rpw + cstep * ch, ch)
            pltpu.sync_copy(idx_hbm.at[pl.ds(base, ch)], idx_v)
            cu = pltpu.async_copy(tu_hbm.at[idx_v], rowsu_v, sem)
            ca = pltpu.async_copy(ta_hbm.at[idx_v], rowsa_v, sem)
            cu.wait()
            ca.wait()
            pltpu.sync_copy(rowsu_v, outu_hbm.at[pl.ds(base, ch)])
            pltpu.sync_copy(rowsa_v, outa_hbm.at[pl.ds(base, ch)])
            return carry

        lax.fori_loop(0, nch, body, 0)

    return sc_gather


def kernel(pc1, feat1, flow, cost, W_rw1, b_rw1, W_rw2, b_rw2, W_f1, b_f1,
           g_bn1, be_bn1, W_f2, b_f2, g_bn2, be_bn2, W_f3, b_f3, W_fc, b_fc):
    f32 = jnp.float32
    B, _, N = pc1.shape
    D = feat1.shape[1]
    CC = cost.shape[1]
    K = NK
    BNr = B * N
    BNK = BNr * K
    C1 = W_f1.shape[0]
    R = W_rw1.shape[0]
    C2 = W_f2.shape[0]
    C3 = W_f3.shape[0]
    WIDTH = C1 + R

    pc1t = pc1.transpose(0, 2, 1).reshape(BNr, 3)
    feat1t = feat1.transpose(0, 2, 1).reshape(BNr, D)
    flowt = flow.transpose(0, 2, 1).reshape(BNr, 3)
    costt = cost.transpose(0, 2, 1).reshape(BNr, CC)
    XIN = 3 + D + 3 + CC + 1
    XPAD = 256
    X = jnp.concatenate(
        [pc1t, feat1t, flowt, costt, jnp.ones((BNr, 1), f32),
         jnp.zeros((BNr, XPAD - XIN), f32)], axis=1)

    Wx = W_f1[:, 0:3]
    Wgf = W_f1[:, 3:3 + D]
    Wcf = W_f1[:, 3 + D:3 + 2 * D]
    Wgw = W_f1[:, 3 + 2 * D:6 + 2 * D]
    Wcw = W_f1[:, 6 + 2 * D:9 + 2 * D]
    Wgc = W_f1[:, 9 + 2 * D:9 + 2 * D + CC]
    A_rw = W_rw1[:, 0:3] + W_rw1[:, 3:6]
    B_rw = W_rw1[:, 6:9] - W_rw1[:, 0:3]
    W_TU = jnp.concatenate([
        Wx.T, Wgf.T, Wgw.T, Wgc.T, jnp.zeros((1 + XPAD - XIN, C1), f32),
    ], axis=0)
    W_TA = jnp.concatenate([
        A_rw.T, jnp.zeros((XPAD - 3, R), f32),
    ], axis=0)
    W_CU = jnp.concatenate([
        -Wx.T, Wcf.T, Wcw.T, jnp.zeros((CC, C1), f32), b_f1[None, :],
        jnp.zeros((XPAD - XIN, C1), f32),
    ], axis=0)
    W_CA = jnp.concatenate([
        B_rw.T, jnp.zeros((XIN - 4, R), f32), b_rw1[None, :],
        jnp.zeros((XPAD - XIN, R), f32),
    ], axis=0)

    PT = 512
    full = lambda i: (0, 0)
    TU, TA, CU, CA = pl.pallas_call(
        _precompute_body,
        grid=(BNr // PT,),
        in_specs=[
            pl.BlockSpec((PT, XPAD), lambda i: (i, 0)),
            pl.BlockSpec((XPAD, C1), full),
            pl.BlockSpec((XPAD, R), full),
            pl.BlockSpec((XPAD, C1), full),
            pl.BlockSpec((XPAD, R), full),
        ],
        out_specs=[
            pl.BlockSpec((PT, C1), lambda i: (i, 0)),
            pl.BlockSpec((PT, R), lambda i: (i, 0)),
            pl.BlockSpec((PT, C1), lambda i: (i, 0)),
            pl.BlockSpec((PT, R), lambda i: (i, 0)),
        ],
        out_shape=[
            jax.ShapeDtypeStruct((BNr, C1), f32),
            jax.ShapeDtypeStruct((BNr, R), f32),
            jax.ShapeDtypeStruct((BNr, C1), f32),
            jax.ShapeDtypeStruct((BNr, R), f32),
        ],
    )(X, W_TU, W_TA, W_CU, W_CA)

    TN = 128
    pc3 = pc1t.reshape(B, N, 3)
    NK1 = N * K
    CH = 128
    gather_fn = _make_sc_gather(NK1, C1, R, CH)
    kidx_parts = []
    gu_parts, ga_parts = [], []
    for b in range(B):
        kb = pl.pallas_call(
            functools.partial(_topk_body, tn=TN, n=N, k=K),
            grid=(1, N // TN),
            in_specs=[
                pl.BlockSpec((1, TN, 3), lambda _, i: (0, i, 0)),
                pl.BlockSpec((1, 3, N), lambda _, i: (0, 0, 0)),
            ],
            out_specs=pl.BlockSpec((1, TN, K), lambda _, i: (0, i, 0)),
            out_shape=jax.ShapeDtypeStruct((1, N, K), jnp.int32),
            scratch_shapes=[pltpu.VMEM((TN, N), f32)],
        )(pc3[b:b + 1], pc1[b:b + 1])
        kidx_parts.append(kb.reshape(NK1) + b * N)
    for b in range(B):
        gu, ga = gather_fn(TU, TA, kidx_parts[b])
        gu_parts.append(gu.reshape(N, K, C1))
        ga_parts.append(ga.reshape(N, K, R))

    TNN = 128
    grid_r = N // TNN
    gu_spec = pl.BlockSpec((TNN, K, C1), lambda i: (i, 0, 0))
    ga_spec = pl.BlockSpec((TNN, K, R), lambda i: (i, 0, 0))
    cu_spec = pl.BlockSpec((TNN, C1), lambda i: (i, 0))
    ca_spec = pl.BlockSpec((TNN, R), lambda i: (i, 0))
    s1_spec = pl.BlockSpec((2, C1), lambda i: (0, 0))
    CU2 = CU.reshape(B, N, C1)
    CA2 = CA.reshape(B, N, R)
    stats1 = sum(
        pl.pallas_call(
            functools.partial(_stats1_body, tnn=TNN, k=K, c1=C1),
            grid=(grid_r,),
            in_specs=[gu_spec, cu_spec],
            out_specs=s1_spec,
            out_shape=jax.ShapeDtypeStruct((2, C1), f32),
        )(gu_parts[b], CU2[b])
        for b in range(B))

    s2_spec = pl.BlockSpec((2, C2), lambda i: (0, 0))
    h2_parts, s2_parts = [], []
    for b in range(B):
        h2b, s2b = pl.pallas_call(
            functools.partial(_m2_body, tnn=TNN, k=K, c1=C1, cnt=float(BNK)),
            grid=(grid_r,),
            in_specs=[
                gu_spec, cu_spec, s1_spec,
                pl.BlockSpec((1, C1), full),
                pl.BlockSpec((1, C1), full),
                pl.BlockSpec((C1, C2), full),
                pl.BlockSpec((1, C2), full),
            ],
            out_specs=[
                pl.BlockSpec((TNN * K, C2), lambda i: (i, 0)),
                s2_spec,
            ],
            out_shape=[
                jax.ShapeDtypeStruct((NK1, C2), f32),
                jax.ShapeDtypeStruct((2, C2), f32),
            ],
        )(gu_parts[b], CU2[b], stats1, g_bn1.reshape(1, C1),
          be_bn1.reshape(1, C1), W_f2.T, b_f2.reshape(1, C2))
        h2_parts.append(h2b)
        s2_parts.append(s2b)
    stats2 = sum(s2_parts)

    flow2 = flowt.reshape(B, N, 3)
    out_parts = []
    for b in range(B):
        outp = pl.pallas_call(
            functools.partial(_m3_body, tnn=TNN, k=K, c1=C1, cnt=float(BNK)),
            grid=(grid_r,),
            in_specs=[
                pl.BlockSpec((TNN * K, C2), lambda i: (i, 0)),
                ga_spec, ca_spec, s2_spec,
                pl.BlockSpec((1, C2), full),
                pl.BlockSpec((1, C2), full),
                pl.BlockSpec((C2, C3), full),
                pl.BlockSpec((1, C3), full),
                pl.BlockSpec((1, 1, R), lambda i: (0, 0, 0)),
                pl.BlockSpec((1, 1, C3), lambda i: (0, 0, 0)),
                pl.BlockSpec((1, 1), full),
                pl.BlockSpec((C3, 3), full),
                pl.BlockSpec((1, 3), full),
                pl.BlockSpec((TNN, 3), lambda i: (i, 0)),
            ],
            out_specs=pl.BlockSpec((TNN, 3), lambda i: (i, 0)),
            out_shape=jax.ShapeDtypeStruct((N, 3), f32),
        )(h2_parts[b], ga_parts[b], CA2[b], stats2,
          g_bn2.reshape(1, C2), be_bn2.reshape(1, C2),
          W_f3.T, b_f3.reshape(1, C3), W_rw2[:, :R].reshape(1, 1, R),
          W_rw2[:, R:].reshape(1, 1, C3), b_rw2.reshape(1, 1), W_fc.T,
          b_fc.reshape(1, 3), flow2[b])
        out_parts.append(outp)

    return jnp.stack(out_parts).transpose(0, 2, 1)


# speedup vs baseline: 14.3269x; 1.0852x over previous
import functools

import jax
import jax.numpy as jnp
from jax import lax
from jax.experimental import pallas as pl
from jax.experimental.pallas import tpu as pltpu
from jax.experimental.pallas import tpu_sc as plsc

NK = 16
BN_EPS = 1e-5
BIG = 3.0e38


def _precompute_body(x_ref, wtu_ref, wta_ref, wcu_ref, wca_ref,
                     tu_ref, ta_ref, cu_ref, ca_ref):
    x = x_ref[...]
    tu_ref[...] = jnp.dot(x, wtu_ref[...], preferred_element_type=jnp.float32)
    ta_ref[...] = jnp.dot(x, wta_ref[...], preferred_element_type=jnp.float32)
    cu_ref[...] = jnp.dot(x, wcu_ref[...], preferred_element_type=jnp.float32)
    ca_ref[...] = jnp.dot(x, wca_ref[...], preferred_element_type=jnp.float32)


def _topk_body(rows_ref, cols_ref, out_ref, d_ref, p_ref, *, tn, n, k):
    b = pl.program_id(0)
    rows = rows_ref[0]
    cols = cols_ref[0]
    colsq = jnp.sum(cols * cols, axis=0, keepdims=True)
    d_ref[...] = colsq - 2.0 * jnp.dot(rows, cols,
                                       preferred_element_type=jnp.float32)
    iota_c = lax.broadcasted_iota(jnp.int32, (tn, n), 1)
    iota_k = lax.broadcasted_iota(jnp.int32, (tn, k), 1)

    bits = lax.bitcast_convert_type(d_ref[...], jnp.int32)
    keys = bits ^ ((bits >> 31) & jnp.int32(0x7FFFFFFF))
    p_ref[...] = (keys & jnp.int32(~0xFFF)) | iota_c

    def body(t, carry):
        v, vbuf, v16, v17 = carry
        p = p_ref[...]
        vnew = jnp.min(jnp.where(p > v, p, jnp.int32(0x7FFFFFFF)),
                       axis=1, keepdims=True)
        m = (iota_k == t).astype(jnp.int32)
        vbuf = vbuf * (1 - m) + vnew * m
        m16 = (t == k - 1).astype(jnp.int32)
        v16 = v16 * (1 - m16) + vnew * m16
        m17 = (t == k).astype(jnp.int32)
        v17 = v17 * (1 - m17) + vnew * m17
        return vnew, vbuf, v16, v17

    _, vbuf, v16, v17 = lax.fori_loop(
        0, k + 1, body,
        (jnp.full((tn, 1), jnp.int32(-0x80000000), jnp.int32),
         jnp.zeros((tn, k), jnp.int32),
         jnp.zeros((tn, 1), jnp.int32),
         jnp.zeros((tn, 1), jnp.int32)))

    c16 = v16 & jnp.int32(0xFFF)
    c17 = v17 & jnp.int32(0xFFF)
    d = d_ref[...]
    e16 = jnp.min(jnp.where(iota_c == c16, d, BIG), axis=1, keepdims=True)
    e17 = jnp.min(jnp.where(iota_c == c17, d, BIG), axis=1, keepdims=True)
    tie = (v16 & jnp.int32(~0xFFF)) == (v17 & jnp.int32(~0xFFF))
    swap = tie & ((e17 < e16) | ((e17 == e16) & (c17 < c16)))
    obuf = vbuf & jnp.int32(0xFFF)
    mfix = ((iota_k == k - 1) & swap).astype(jnp.int32)
    obuf = obuf * (1 - mfix) + c17 * mfix
    out_ref[0] = obuf + b * n


def _stats1_body(g_ref, cb_ref, s_ref, *, tnn, k, c1):
    h = g_ref[...] + cb_ref[...][:, None, :]
    hr = h.reshape(tnn * k, c1)
    st = jnp.stack([jnp.sum(hr, axis=0), jnp.sum(hr * hr, axis=0)])

    @pl.when(pl.program_id(0) == 0)
    def _():
        s_ref[...] = st

    @pl.when(pl.program_id(0) != 0)
    def _():
        s_ref[...] += st


def _bn_affine(s_ref, g_ref, be_ref, cnt):
    mu = s_ref[0:1, :] * (1.0 / cnt)
    var = s_ref[1:2, :] * (1.0 / cnt) - mu * mu
    a = g_ref[...] * lax.rsqrt(var + BN_EPS)
    c = be_ref[...] - mu * a
    return a, c


def _m2_body(g_ref, cb_ref, s_ref, g1_ref, be1_ref, w2_ref, b2_ref,
             h2_ref, s2_ref, *, tnn, k, c1, cnt):
    a, c = _bn_affine(s_ref, g1_ref, be1_ref, cnt)
    h = g_ref[...] + cb_ref[...][:, None, :]
    y = a * h.reshape(tnn * k, c1) + c
    y = jnp.where(y >= 0, y, 0.1 * y)
    h2 = jnp.dot(y, w2_ref[...], preferred_element_type=jnp.float32)
    h2 = h2 + b2_ref[...]
    h2_ref[...] = h2
    st = jnp.stack([jnp.sum(h2, axis=0), jnp.sum(h2 * h2, axis=0)])

    @pl.when(pl.program_id(0) == 0)
    def _():
        s2_ref[...] = st

    @pl.when(pl.program_id(0) != 0)
    def _():
        s2_ref[...] += st


def _m3_body(h2_ref, g_ref, cb_ref, s_ref, g2_ref, be2_ref, w3_ref, b3_ref,
             w2a_ref, w2b_ref, brw2_ref, wfc_ref, bfc_ref, fl_ref, o_ref,
             *, tnn, k, c1, cnt):
    a, c = _bn_affine(s_ref, g2_ref, be2_ref, cnt)
    y = a * h2_ref[...] + c
    y = jnp.where(y >= 0, y, 0.1 * y)
    reflow = jnp.dot(y, w3_ref[...],
                     preferred_element_type=jnp.float32) + b3_ref[...]
    reflow3 = reflow.reshape(tnn, k, reflow.shape[-1])
    rew3 = g_ref[...] + cb_ref[...][:, None, :]
    logits = jnp.sum(rew3 * w2a_ref[...] + reflow3 * w2b_ref[...], axis=2)
    logits = logits + brw2_ref[...]
    m = jnp.max(logits, axis=1, keepdims=True)
    e = jnp.exp(logits - m)
    w = e / jnp.sum(e, axis=1, keepdims=True)
    rwf = jnp.sum(reflow3 * w[:, :, None], axis=1)
    o = jnp.dot(rwf, wfc_ref[...],
                preferred_element_type=jnp.float32) + bfc_ref[...]
    o_ref[...] = o + fl_ref[...]


def _make_sc_gather(bnk, wu, wa, ch):
    nw = 32
    rpw = bnk // nw
    nch = rpw // ch
    mesh = plsc.VectorSubcoreMesh(core_axis_name="c", subcore_axis_name="s")

    @functools.partial(
        pl.kernel,
        mesh=mesh,
        out_type=(jax.ShapeDtypeStruct((bnk, wu), jnp.float32),
                  jax.ShapeDtypeStruct((bnk, wa), jnp.float32)),
        scratch_types=[
            pltpu.VMEM((ch,), jnp.int32),
            pltpu.VMEM((ch, wu), jnp.float32),
            pltpu.VMEM((ch, wa), jnp.float32),
            pltpu.SemaphoreType.DMA,
        ],
        compiler_params=pltpu.CompilerParams(use_tc_tiling_on_sc=False),
    )
    def sc_gather(tu_hbm, ta_hbm, idx_hbm, outu_hbm, outa_hbm,
                  idx_v, rowsu_v, rowsa_v, sem):
        wid = lax.axis_index("s") * 2 + lax.axis_index("c")

        def body(cstep, carry):
            base = pl.multiple_of(wid * rpw + cstep * ch, ch)
            pltpu.sync_copy(idx_hbm.at[pl.ds(base, ch)], idx_v)
            cu = pltpu.async_copy(tu_hbm.at[idx_v], rowsu_v, sem)
            ca = pltpu.async_copy(ta_hbm.at[idx_v], rowsa_v, sem)
            cu.wait()
            ca.wait()
            pltpu.sync_copy(rowsu_v, outu_hbm.at[pl.ds(base, ch)])
            pltpu.sync_copy(rowsa_v, outa_hbm.at[pl.ds(base, ch)])
            return carry

        lax.fori_loop(0, nch, body, 0)

    return sc_gather


def kernel(pc1, feat1, flow, cost, W_rw1, b_rw1, W_rw2, b_rw2, W_f1, b_f1,
           g_bn1, be_bn1, W_f2, b_f2, g_bn2, be_bn2, W_f3, b_f3, W_fc, b_fc):
    f32 = jnp.float32
    B, _, N = pc1.shape
    D = feat1.shape[1]
    CC = cost.shape[1]
    K = NK
    BNr = B * N
    BNK = BNr * K
    C1 = W_f1.shape[0]
    R = W_rw1.shape[0]
    C2 = W_f2.shape[0]
    C3 = W_f3.shape[0]
    WIDTH = C1 + R

    pc1t = pc1.transpose(0, 2, 1).reshape(BNr, 3)
    feat1t = feat1.transpose(0, 2, 1).reshape(BNr, D)
    flowt = flow.transpose(0, 2, 1).reshape(BNr, 3)
    costt = cost.transpose(0, 2, 1).reshape(BNr, CC)
    XIN = 3 + D + 3 + CC + 1
    XPAD = 256
    X = jnp.concatenate(
        [pc1t, feat1t, flowt, costt, jnp.ones((BNr, 1), f32),
         jnp.zeros((BNr, XPAD - XIN), f32)], axis=1)

    Wx = W_f1[:, 0:3]
    Wgf = W_f1[:, 3:3 + D]
    Wcf = W_f1[:, 3 + D:3 + 2 * D]
    Wgw = W_f1[:, 3 + 2 * D:6 + 2 * D]
    Wcw = W_f1[:, 6 + 2 * D:9 + 2 * D]
    Wgc = W_f1[:, 9 + 2 * D:9 + 2 * D + CC]
    A_rw = W_rw1[:, 0:3] + W_rw1[:, 3:6]
    B_rw = W_rw1[:, 6:9] - W_rw1[:, 0:3]
    W_TU = jnp.concatenate([
        Wx.T, Wgf.T, Wgw.T, Wgc.T, jnp.zeros((1 + XPAD - XIN, C1), f32),
    ], axis=0)
    W_TA = jnp.concatenate([
        A_rw.T, jnp.zeros((XPAD - 3, R), f32),
    ], axis=0)
    W_CU = jnp.concatenate([
        -Wx.T, Wcf.T, Wcw.T, jnp.zeros((CC, C1), f32), b_f1[None, :],
        jnp.zeros((XPAD - XIN, C1), f32),
    ], axis=0)
    W_CA = jnp.concatenate([
        B_rw.T, jnp.zeros((XIN - 4, R), f32), b_rw1[None, :],
        jnp.zeros((XPAD - XIN, R), f32),
    ], axis=0)

    PT = 512
    full = lambda i: (0, 0)
    TU, TA, CU, CA = pl.pallas_call(
        _precompute_body,
        grid=(BNr // PT,),
        in_specs=[
            pl.BlockSpec((PT, XPAD), lambda i: (i, 0)),
            pl.BlockSpec((XPAD, C1), full),
            pl.BlockSpec((XPAD, R), full),
            pl.BlockSpec((XPAD, C1), full),
            pl.BlockSpec((XPAD, R), full),
        ],
        out_specs=[
            pl.BlockSpec((PT, C1), lambda i: (i, 0)),
            pl.BlockSpec((PT, R), lambda i: (i, 0)),
            pl.BlockSpec((PT, C1), lambda i: (i, 0)),
            pl.BlockSpec((PT, R), lambda i: (i, 0)),
        ],
        out_shape=[
            jax.ShapeDtypeStruct((BNr, C1), f32),
            jax.ShapeDtypeStruct((BNr, R), f32),
            jax.ShapeDtypeStruct((BNr, C1), f32),
            jax.ShapeDtypeStruct((BNr, R), f32),
        ],
    )(X, W_TU, W_TA, W_CU, W_CA)

    TN = 128
    pc3 = pc1t.reshape(B, N, 3)
    NK1 = N * K
    CH = 128
    gather_fn = _make_sc_gather(NK1, C1, R, CH)
    kidx_parts = []
    gu_parts, ga_parts = [], []
    for b in range(B):
        kb = pl.pallas_call(
            functools.partial(_topk_body, tn=TN, n=N, k=K),
            grid=(1, N // TN),
            in_specs=[
                pl.BlockSpec((1, TN, 3), lambda _, i: (0, i, 0)),
                pl.BlockSpec((1, 3, N), lambda _, i: (0, 0, 0)),
            ],
            out_specs=pl.BlockSpec((1, TN, K), lambda _, i: (0, i, 0)),
            out_shape=jax.ShapeDtypeStruct((1, N, K), jnp.int32),
            scratch_shapes=[pltpu.VMEM((TN, N), f32),
                            pltpu.VMEM((TN, N), jnp.int32)],
        )(pc3[b:b + 1], pc1[b:b + 1])
        kidx_parts.append(kb.reshape(NK1) + b * N)
    for b in range(B):
        gu, ga = gather_fn(TU, TA, kidx_parts[b])
        gu_parts.append(gu.reshape(N, K, C1))
        ga_parts.append(ga.reshape(N, K, R))

    TNN = 128
    grid_r = N // TNN
    gu_spec = pl.BlockSpec((TNN, K, C1), lambda i: (i, 0, 0))
    ga_spec = pl.BlockSpec((TNN, K, R), lambda i: (i, 0, 0))
    cu_spec = pl.BlockSpec((TNN, C1), lambda i: (i, 0))
    ca_spec = pl.BlockSpec((TNN, R), lambda i: (i, 0))
    s1_spec = pl.BlockSpec((2, C1), lambda i: (0, 0))
    CU2 = CU.reshape(B, N, C1)
    CA2 = CA.reshape(B, N, R)
    stats1 = sum(
        pl.pallas_call(
            functools.partial(_stats1_body, tnn=TNN, k=K, c1=C1),
            grid=(grid_r,),
            in_specs=[gu_spec, cu_spec],
            out_specs=s1_spec,
            out_shape=jax.ShapeDtypeStruct((2, C1), f32),
        )(gu_parts[b], CU2[b])
        for b in range(B))

    s2_spec = pl.BlockSpec((2, C2), lambda i: (0, 0))
    h2_parts, s2_parts = [], []
    for b in range(B):
        h2b, s2b = pl.pallas_call(
            functools.partial(_m2_body, tnn=TNN, k=K, c1=C1, cnt=float(BNK)),
            grid=(grid_r,),
            in_specs=[
                gu_spec, cu_spec, s1_spec,
                pl.BlockSpec((1, C1), full),
                pl.BlockSpec((1, C1), full),
                pl.BlockSpec((C1, C2), full),
                pl.BlockSpec((1, C2), full),
            ],
            out_specs=[
                pl.BlockSpec((TNN * K, C2), lambda i: (i, 0)),
                s2_spec,
            ],
            out_shape=[
                jax.ShapeDtypeStruct((NK1, C2), f32),
                jax.ShapeDtypeStruct((2, C2), f32),
            ],
        )(gu_parts[b], CU2[b], stats1, g_bn1.reshape(1, C1),
          be_bn1.reshape(1, C1), W_f2.T, b_f2.reshape(1, C2))
        h2_parts.append(h2b)
        s2_parts.append(s2b)
    stats2 = sum(s2_parts)

    flow2 = flowt.reshape(B, N, 3)
    out_parts = []
    for b in range(B):
        outp = pl.pallas_call(
            functools.partial(_m3_body, tnn=TNN, k=K, c1=C1, cnt=float(BNK)),
            grid=(grid_r,),
            in_specs=[
                pl.BlockSpec((TNN * K, C2), lambda i: (i, 0)),
                ga_spec, ca_spec, s2_spec,
                pl.BlockSpec((1, C2), full),
                pl.BlockSpec((1, C2), full),
                pl.BlockSpec((C2, C3), full),
                pl.BlockSpec((1, C3), full),
                pl.BlockSpec((1, 1, R), lambda i: (0, 0, 0)),
                pl.BlockSpec((1, 1, C3), lambda i: (0, 0, 0)),
                pl.BlockSpec((1, 1), full),
                pl.BlockSpec((C3, 3), full),
                pl.BlockSpec((1, 3), full),
                pl.BlockSpec((TNN, 3), lambda i: (i, 0)),
            ],
            out_specs=pl.BlockSpec((TNN, 3), lambda i: (i, 0)),
            out_shape=jax.ShapeDtypeStruct((N, 3), f32),
        )(h2_parts[b], ga_parts[b], CA2[b], stats2,
          g_bn2.reshape(1, C2), be_bn2.reshape(1, C2),
          W_f3.T, b_f3.reshape(1, C3), W_rw2[:, :R].reshape(1, 1, R),
          W_rw2[:, R:].reshape(1, 1, C3), b_rw2.reshape(1, 1), W_fc.T,
          b_fc.reshape(1, 3), flow2[b])
        out_parts.append(outp)

    return jnp.stack(out_parts).transpose(0, 2, 1)


# speedup vs baseline: 16.0001x; 1.1168x over previous
import functools

import jax
import jax.numpy as jnp
from jax import lax
from jax.experimental import pallas as pl
from jax.experimental.pallas import tpu as pltpu
from jax.experimental.pallas import tpu_sc as plsc

NK = 16
BN_EPS = 1e-5
BIG = 3.0e38


def _precompute_body(x_ref, wtu_ref, wta_ref, wcu_ref, wca_ref,
                     tu_ref, ta_ref, cu_ref, ca_ref):
    x = x_ref[...]
    tu_ref[...] = jnp.dot(x, wtu_ref[...], preferred_element_type=jnp.float32)
    ta_ref[...] = jnp.dot(x, wta_ref[...], preferred_element_type=jnp.float32)
    cu_ref[...] = jnp.dot(x, wcu_ref[...], preferred_element_type=jnp.float32)
    ca_ref[...] = jnp.dot(x, wca_ref[...], preferred_element_type=jnp.float32)


def _topk_body(rows_ref, cols_ref, out_ref, d_ref, p_ref, *, tn, n, k):
    b = pl.program_id(0)
    rows = rows_ref[0]
    cols = cols_ref[0]
    colsq = jnp.sum(cols * cols, axis=0, keepdims=True)
    d_ref[...] = colsq - 2.0 * jnp.dot(rows, cols,
                                       preferred_element_type=jnp.float32)
    iota_c = lax.broadcasted_iota(jnp.int32, (tn, n), 1)
    iota_k = lax.broadcasted_iota(jnp.int32, (tn, k), 1)

    bits = lax.bitcast_convert_type(d_ref[...], jnp.int32)
    keys = bits ^ ((bits >> 31) & jnp.int32(0x7FFFFFFF))
    p_ref[...] = (keys & jnp.int32(~0xFFF)) | iota_c

    def body(t, carry):
        v, vbuf, v16, v17 = carry
        p = p_ref[...]
        vnew = jnp.min(jnp.where(p > v, p, jnp.int32(0x7FFFFFFF)),
                       axis=1, keepdims=True)
        m = (iota_k == t).astype(jnp.int32)
        vbuf = vbuf * (1 - m) + vnew * m
        m16 = (t == k - 1).astype(jnp.int32)
        v16 = v16 * (1 - m16) + vnew * m16
        m17 = (t == k).astype(jnp.int32)
        v17 = v17 * (1 - m17) + vnew * m17
        return vnew, vbuf, v16, v17

    _, vbuf, v16, v17 = lax.fori_loop(
        0, k + 1, body,
        (jnp.full((tn, 1), jnp.int32(-0x80000000), jnp.int32),
         jnp.zeros((tn, k), jnp.int32),
         jnp.zeros((tn, 1), jnp.int32),
         jnp.zeros((tn, 1), jnp.int32)))

    c16 = v16 & jnp.int32(0xFFF)
    c17 = v17 & jnp.int32(0xFFF)
    d = d_ref[...]
    e16 = jnp.min(jnp.where(iota_c == c16, d, BIG), axis=1, keepdims=True)
    e17 = jnp.min(jnp.where(iota_c == c17, d, BIG), axis=1, keepdims=True)
    tie = (v16 & jnp.int32(~0xFFF)) == (v17 & jnp.int32(~0xFFF))
    swap = tie & ((e17 < e16) | ((e17 == e16) & (c17 < c16)))
    obuf = vbuf & jnp.int32(0xFFF)
    mfix = ((iota_k == k - 1) & swap).astype(jnp.int32)
    obuf = obuf * (1 - mfix) + c17 * mfix
    out_ref[0] = obuf + b * n


def _stats1_body(g_ref, cb_ref, s_ref, *, tnn, k, c1):
    h = g_ref[...] + cb_ref[...][:, None, :]
    hr = h.reshape(tnn * k, c1)
    st = jnp.stack([jnp.sum(hr, axis=0), jnp.sum(hr * hr, axis=0)])

    @pl.when(pl.program_id(0) == 0)
    def _():
        s_ref[...] = st

    @pl.when(pl.program_id(0) != 0)
    def _():
        s_ref[...] += st


def _bn_affine(s_ref, g_ref, be_ref, cnt):
    mu = s_ref[0:1, :] * (1.0 / cnt)
    var = s_ref[1:2, :] * (1.0 / cnt) - mu * mu
    a = g_ref[...] * lax.rsqrt(var + BN_EPS)
    c = be_ref[...] - mu * a
    return a, c


def _m2_body(g_ref, cb_ref, s_ref, g1_ref, be1_ref, w2_ref, b2_ref,
             h2_ref, s2_ref, *, tnn, k, c1, cnt):
    a, c = _bn_affine(s_ref, g1_ref, be1_ref, cnt)
    h = g_ref[...] + cb_ref[...][:, None, :]
    y = a * h.reshape(tnn * k, c1) + c
    y = jnp.where(y >= 0, y, 0.1 * y)
    h2 = jnp.dot(y, w2_ref[...], preferred_element_type=jnp.float32)
    h2 = h2 + b2_ref[...]
    h2_ref[...] = h2.astype(jnp.bfloat16)
    st = jnp.stack([jnp.sum(h2, axis=0), jnp.sum(h2 * h2, axis=0)])

    @pl.when(pl.program_id(0) == 0)
    def _():
        s2_ref[...] = st

    @pl.when(pl.program_id(0) != 0)
    def _():
        s2_ref[...] += st


def _m3_body(h2_ref, g_ref, cb_ref, s_ref, g2_ref, be2_ref, w3_ref, b3_ref,
             w2a_ref, w2b_ref, brw2_ref, wfc_ref, bfc_ref, fl_ref, o_ref,
             *, tnn, k, c1, cnt):
    a, c = _bn_affine(s_ref, g2_ref, be2_ref, cnt)
    y = a * h2_ref[...].astype(jnp.float32) + c
    y = jnp.where(y >= 0, y, 0.1 * y)
    reflow = jnp.dot(y, w3_ref[...],
                     preferred_element_type=jnp.float32) + b3_ref[...]
    reflow3 = reflow.reshape(tnn, k, reflow.shape[-1])
    rew3 = g_ref[...] + cb_ref[...][:, None, :]
    logits = jnp.sum(rew3 * w2a_ref[...] + reflow3 * w2b_ref[...], axis=2)
    logits = logits + brw2_ref[...]
    m = jnp.max(logits, axis=1, keepdims=True)
    e = jnp.exp(logits - m)
    w = e / jnp.sum(e, axis=1, keepdims=True)
    rwf = jnp.sum(reflow3 * w[:, :, None], axis=1)
    o = jnp.dot(rwf, wfc_ref[...],
                preferred_element_type=jnp.float32) + bfc_ref[...]
    o_ref[...] = o + fl_ref[...]


def _make_sc_gather(bnk, wu, wa, ch):
    nw = 32
    rpw = bnk // nw
    nch = rpw // ch
    mesh = plsc.VectorSubcoreMesh(core_axis_name="c", subcore_axis_name="s")

    @functools.partial(
        pl.kernel,
        mesh=mesh,
        out_type=(jax.ShapeDtypeStruct((bnk, wu), jnp.float32),
                  jax.ShapeDtypeStruct((bnk, wa), jnp.float32)),
        scratch_types=[
            pltpu.VMEM((ch,), jnp.int32),
            pltpu.VMEM((ch, wu), jnp.float32),
            pltpu.VMEM((ch, wa), jnp.float32),
            pltpu.SemaphoreType.DMA,
        ],
        compiler_params=pltpu.CompilerParams(use_tc_tiling_on_sc=False),
    )
    def sc_gather(tu_hbm, ta_hbm, idx_hbm, outu_hbm, outa_hbm,
                  idx_v, rowsu_v, rowsa_v, sem):
        wid = lax.axis_index("s") * 2 + lax.axis_index("c")

        def body(cstep, carry):
            base = pl.multiple_of(wid * rpw + cstep * ch, ch)
            pltpu.sync_copy(idx_hbm.at[pl.ds(base, ch)], idx_v)
            cu = pltpu.async_copy(tu_hbm.at[idx_v], rowsu_v, sem)
            ca = pltpu.async_copy(ta_hbm.at[idx_v], rowsa_v, sem)
            cu.wait()
            ca.wait()
            pltpu.sync_copy(rowsu_v, outu_hbm.at[pl.ds(base, ch)])
            pltpu.sync_copy(rowsa_v, outa_hbm.at[pl.ds(base, ch)])
            return carry

        lax.fori_loop(0, nch, body, 0)

    return sc_gather


def kernel(pc1, feat1, flow, cost, W_rw1, b_rw1, W_rw2, b_rw2, W_f1, b_f1,
           g_bn1, be_bn1, W_f2, b_f2, g_bn2, be_bn2, W_f3, b_f3, W_fc, b_fc):
    f32 = jnp.float32
    B, _, N = pc1.shape
    D = feat1.shape[1]
    CC = cost.shape[1]
    K = NK
    BNr = B * N
    BNK = BNr * K
    C1 = W_f1.shape[0]
    R = W_rw1.shape[0]
    C2 = W_f2.shape[0]
    C3 = W_f3.shape[0]
    WIDTH = C1 + R

    pc1t = pc1.transpose(0, 2, 1).reshape(BNr, 3)
    feat1t = feat1.transpose(0, 2, 1).reshape(BNr, D)
    flowt = flow.transpose(0, 2, 1).reshape(BNr, 3)
    costt = cost.transpose(0, 2, 1).reshape(BNr, CC)
    XIN = 3 + D + 3 + CC + 1
    XPAD = 256
    X = jnp.concatenate(
        [pc1t, feat1t, flowt, costt, jnp.ones((BNr, 1), f32),
         jnp.zeros((BNr, XPAD - XIN), f32)], axis=1)

    Wx = W_f1[:, 0:3]
    Wgf = W_f1[:, 3:3 + D]
    Wcf = W_f1[:, 3 + D:3 + 2 * D]
    Wgw = W_f1[:, 3 + 2 * D:6 + 2 * D]
    Wcw = W_f1[:, 6 + 2 * D:9 + 2 * D]
    Wgc = W_f1[:, 9 + 2 * D:9 + 2 * D + CC]
    A_rw = W_rw1[:, 0:3] + W_rw1[:, 3:6]
    B_rw = W_rw1[:, 6:9] - W_rw1[:, 0:3]
    W_TU = jnp.concatenate([
        Wx.T, Wgf.T, Wgw.T, Wgc.T, jnp.zeros((1 + XPAD - XIN, C1), f32),
    ], axis=0)
    W_TA = jnp.concatenate([
        A_rw.T, jnp.zeros((XPAD - 3, R), f32),
    ], axis=0)
    W_CU = jnp.concatenate([
        -Wx.T, Wcf.T, Wcw.T, jnp.zeros((CC, C1), f32), b_f1[None, :],
        jnp.zeros((XPAD - XIN, C1), f32),
    ], axis=0)
    W_CA = jnp.concatenate([
        B_rw.T, jnp.zeros((XIN - 4, R), f32), b_rw1[None, :],
        jnp.zeros((XPAD - XIN, R), f32),
    ], axis=0)

    PT = 512
    full = lambda i: (0, 0)
    TU, TA, CU, CA = pl.pallas_call(
        _precompute_body,
        grid=(BNr // PT,),
        in_specs=[
            pl.BlockSpec((PT, XPAD), lambda i: (i, 0)),
            pl.BlockSpec((XPAD, C1), full),
            pl.BlockSpec((XPAD, R), full),
            pl.BlockSpec((XPAD, C1), full),
            pl.BlockSpec((XPAD, R), full),
        ],
        out_specs=[
            pl.BlockSpec((PT, C1), lambda i: (i, 0)),
            pl.BlockSpec((PT, R), lambda i: (i, 0)),
            pl.BlockSpec((PT, C1), lambda i: (i, 0)),
            pl.BlockSpec((PT, R), lambda i: (i, 0)),
        ],
        out_shape=[
            jax.ShapeDtypeStruct((BNr, C1), f32),
            jax.ShapeDtypeStruct((BNr, R), f32),
            jax.ShapeDtypeStruct((BNr, C1), f32),
            jax.ShapeDtypeStruct((BNr, R), f32),
        ],
    )(X, W_TU, W_TA, W_CU, W_CA)

    TN = 256
    pc3 = pc1t.reshape(B, N, 3)
    NK1 = N * K
    CH = 128
    gather_fn = _make_sc_gather(NK1, C1, R, CH)
    kidx_parts = []
    gu_parts, ga_parts = [], []
    for b in range(B):
        kb = pl.pallas_call(
            functools.partial(_topk_body, tn=TN, n=N, k=K),
            grid=(1, N // TN),
            in_specs=[
                pl.BlockSpec((1, TN, 3), lambda _, i: (0, i, 0)),
                pl.BlockSpec((1, 3, N), lambda _, i: (0, 0, 0)),
            ],
            out_specs=pl.BlockSpec((1, TN, K), lambda _, i: (0, i, 0)),
            out_shape=jax.ShapeDtypeStruct((1, N, K), jnp.int32),
            scratch_shapes=[pltpu.VMEM((TN, N), f32),
                            pltpu.VMEM((TN, N), jnp.int32)],
        )(pc3[b:b + 1], pc1[b:b + 1])
        kidx_parts.append(kb.reshape(NK1) + b * N)
    for b in range(B):
        gu, ga = gather_fn(TU, TA, kidx_parts[b])
        gu_parts.append(gu.reshape(N, K, C1))
        ga_parts.append(ga.reshape(N, K, R))

    TNN = 128
    grid_r = N // TNN
    gu_spec = pl.BlockSpec((TNN, K, C1), lambda i: (i, 0, 0))
    ga_spec = pl.BlockSpec((TNN, K, R), lambda i: (i, 0, 0))
    cu_spec = pl.BlockSpec((TNN, C1), lambda i: (i, 0))
    ca_spec = pl.BlockSpec((TNN, R), lambda i: (i, 0))
    s1_spec = pl.BlockSpec((2, C1), lambda i: (0, 0))
    CU2 = CU.reshape(B, N, C1)
    CA2 = CA.reshape(B, N, R)
    stats1 = sum(
        pl.pallas_call(
            functools.partial(_stats1_body, tnn=TNN, k=K, c1=C1),
            grid=(grid_r,),
            in_specs=[gu_spec, cu_spec],
            out_specs=s1_spec,
            out_shape=jax.ShapeDtypeStruct((2, C1), f32),
        )(gu_parts[b], CU2[b])
        for b in range(B))

    s2_spec = pl.BlockSpec((2, C2), lambda i: (0, 0))
    h2_parts, s2_parts = [], []
    for b in range(B):
        h2b, s2b = pl.pallas_call(
            functools.partial(_m2_body, tnn=TNN, k=K, c1=C1, cnt=float(BNK)),
            grid=(grid_r,),
            in_specs=[
                gu_spec, cu_spec, s1_spec,
                pl.BlockSpec((1, C1), full),
                pl.BlockSpec((1, C1), full),
                pl.BlockSpec((C1, C2), full),
                pl.BlockSpec((1, C2), full),
            ],
            out_specs=[
                pl.BlockSpec((TNN * K, C2), lambda i: (i, 0)),
                s2_spec,
            ],
            out_shape=[
                jax.ShapeDtypeStruct((NK1, C2), jnp.bfloat16),
                jax.ShapeDtypeStruct((2, C2), f32),
            ],
        )(gu_parts[b], CU2[b], stats1, g_bn1.reshape(1, C1),
          be_bn1.reshape(1, C1), W_f2.T, b_f2.reshape(1, C2))
        h2_parts.append(h2b)
        s2_parts.append(s2b)
    stats2 = sum(s2_parts)

    flow2 = flowt.reshape(B, N, 3)
    out_parts = []
    for b in range(B):
        outp = pl.pallas_call(
            functools.partial(_m3_body, tnn=TNN, k=K, c1=C1, cnt=float(BNK)),
            grid=(grid_r,),
            in_specs=[
                pl.BlockSpec((TNN * K, C2), lambda i: (i, 0)),
                ga_spec, ca_spec, s2_spec,
                pl.BlockSpec((1, C2), full),
                pl.BlockSpec((1, C2), full),
                pl.BlockSpec((C2, C3), full),
                pl.BlockSpec((1, C3), full),
                pl.BlockSpec((1, 1, R), lambda i: (0, 0, 0)),
                pl.BlockSpec((1, 1, C3), lambda i: (0, 0, 0)),
                pl.BlockSpec((1, 1), full),
                pl.BlockSpec((C3, 3), full),
                pl.BlockSpec((1, 3), full),
                pl.BlockSpec((TNN, 3), lambda i: (i, 0)),
            ],
            out_specs=pl.BlockSpec((TNN, 3), lambda i: (i, 0)),
            out_shape=jax.ShapeDtypeStruct((N, 3), f32),
        )(h2_parts[b], ga_parts[b], CA2[b], stats2,
          g_bn2.reshape(1, C2), be_bn2.reshape(1, C2),
          W_f3.T, b_f3.reshape(1, C3), W_rw2[:, :R].reshape(1, 1, R),
          W_rw2[:, R:].reshape(1, 1, C3), b_rw2.reshape(1, 1), W_fc.T,
          b_fc.reshape(1, 3), flow2[b])
        out_parts.append(outp)

    return jnp.stack(out_parts).transpose(0, 2, 1)
